# Initial kernel scaffold; baseline (speedup 1.0000x reference)
import jax
import jax.numpy as jnp
from jax.experimental import pallas as pl


def kernel(h, x, mess_graph, W_z_w, W_z_b, W_r_w, U_r_w, U_r_b, W_h_w, W_h_b):
    raise NotImplementedError("write your pallas kernel here")


# speedup vs baseline: 4.0857x; 4.0857x over previous
import functools

import jax
import jax.numpy as jnp
from jax import lax
from jax.experimental import pallas as pl
from jax.experimental.pallas import tpu as pltpu
from jax.experimental.pallas import tpu_sc as plsc

N = 160000
MAX_NEI = 8
INPUT = 128
HIDDEN = 128
DEPTH = 3

NC = 2
NS = 16
NW = NC * NS
E = N * MAX_NEI
PER_W = E // NW
C = 80
NCHUNK = PER_W // C


def _sc_gather_body(h_hbm, idx_hbm, out_hbm, idx_v, rows_v, gsem0, gsem1):
    wid = lax.axis_index("s") * NC + lax.axis_index("c")
    base = wid * PER_W
    pltpu.sync_copy(idx_hbm.at[wid], idx_v)
    gsems = (gsem0, gsem1)

    def start(ci, b):
        pltpu.async_copy(h_hbm.at[idx_v.at[ci]], rows_v.at[b], gsems[b])

    def finish(ci, b):
        pltpu.make_async_copy(h_hbm.at[idx_v.at[ci]], rows_v.at[b], gsems[b]).wait()
        pltpu.sync_copy(rows_v.at[b], out_hbm.at[pl.ds(base + ci * C, C)])

    start(0, 0)

    def body(k, _):
        ca = 2 * k
        start(ca + 1, 1)
        finish(ca, 0)

        @pl.when(ca + 2 < NCHUNK)
        def _():
            start(ca + 2, 0)

        finish(ca + 1, 1)
        return 0

    lax.fori_loop(0, NCHUNK // 2, body, 0)


_sc_gather = functools.partial(
    pl.kernel,
    out_type=jax.ShapeDtypeStruct((E, HIDDEN), jnp.float32),
    mesh=plsc.VectorSubcoreMesh(core_axis_name="c", subcore_axis_name="s"),
    scratch_types=[
        pltpu.VMEM((NCHUNK, C), jnp.int32),
        pltpu.VMEM((2, C, HIDDEN), jnp.float32),
        pltpu.SemaphoreType.DMA,
        pltpu.SemaphoreType.DMA,
    ],
)(_sc_gather_body)


T = 640


def _tc_gru_body(x_ref, hnei_ref, wr_ref, ur_ref, urb_ref,
                 wzx_ref, wzh_ref, wzb_ref, whx_ref, whh_ref, whb_ref,
                 out_ref):
    xt = x_ref[...]
    r1 = jnp.dot(xt, wr_ref[...], preferred_element_type=jnp.float32)
    urb = urb_ref[...].reshape(1, HIDDEN)

    sum_h = jnp.zeros((T, HIDDEN), jnp.float32)
    sum_g = jnp.zeros((T, HIDDEN), jnp.float32)
    for j in range(MAX_NEI):
        hj = hnei_ref[j]
        r2 = jnp.dot(hj, ur_ref[...], preferred_element_type=jnp.float32)
        r = jax.nn.sigmoid(r1 + r2 + urb)
        sum_h = sum_h + hj
        sum_g = sum_g + r * hj

    z = jax.nn.sigmoid(
        jnp.dot(xt, wzx_ref[...], preferred_element_type=jnp.float32)
        + jnp.dot(sum_h, wzh_ref[...], preferred_element_type=jnp.float32)
        + wzb_ref[...].reshape(1, HIDDEN))
    pre_h = jnp.tanh(
        jnp.dot(xt, whx_ref[...], preferred_element_type=jnp.float32)
        + jnp.dot(sum_g, whh_ref[...], preferred_element_type=jnp.float32)
        + whb_ref[...].reshape(1, HIDDEN))
    h_new = (1.0 - z) * sum_h + z * pre_h

    row = lax.broadcasted_iota(jnp.int32, (T, HIDDEN), 0) + pl.program_id(0) * T
    out_ref[...] = jnp.where(row == 0, 0.0, h_new)


def _tc_gru(x, hnei, wr, ur, urb, wzx, wzh, wzb, whx, whh, whb):
    wspec = pl.BlockSpec((HIDDEN, HIDDEN), lambda i: (0, 0))
    bspec = pl.BlockSpec((HIDDEN,), lambda i: (0,))
    return pl.pallas_call(
        _tc_gru_body,
        grid=(N // T,),
        in_specs=[
            pl.BlockSpec((T, INPUT), lambda i: (i, 0)),
            pl.BlockSpec((MAX_NEI, T, HIDDEN), lambda i: (0, i, 0)),
            wspec, wspec, bspec, wspec, wspec, bspec, wspec, wspec, bspec,
        ],
        out_specs=pl.BlockSpec((T, HIDDEN), lambda i: (i, 0)),
        out_shape=jax.ShapeDtypeStruct((N, HIDDEN), jnp.float32),
    )(x, hnei, wr, ur, urb, wzx, wzh, wzb, whx, whh, whb)


def kernel(h, x, mess_graph, W_z_w, W_z_b, W_r_w, U_r_w, U_r_b, W_h_w, W_h_b):
    wr = W_r_w.T
    ur = U_r_w.T
    wzx = W_z_w[:, :INPUT].T
    wzh = W_z_w[:, INPUT:].T
    whx = W_h_w[:, :INPUT].T
    whh = W_h_w[:, INPUT:].T
    idx3 = mess_graph.T.reshape(NW, NCHUNK, C)

    for _ in range(DEPTH):
        hnei_flat = _sc_gather(h, idx3)
        hnei = hnei_flat.reshape(MAX_NEI, N, HIDDEN)
        h = _tc_gru(x, hnei, wr, ur, U_r_b, wzx, wzh, W_z_b, whx, whh, W_h_b)
    return h


# speedup vs baseline: 4.3525x; 1.0653x over previous
import functools

import jax
import jax.numpy as jnp
from jax import lax
from jax.experimental import pallas as pl
from jax.experimental.pallas import tpu as pltpu
from jax.experimental.pallas import tpu_sc as plsc

N = 160000
MAX_NEI = 8
INPUT = 128
HIDDEN = 128
DEPTH = 3

NC = 2
NS = 16
NW = NC * NS
E = N * MAX_NEI
PER_W = E // NW
C = 80
NCHUNK = PER_W // C
NBUF = 4


def _sc_gather_body(h_hbm, idx_hbm, out_hbm, idx_v, rows_v, *gsems):
    wid = lax.axis_index("s") * NC + lax.axis_index("c")
    base = wid * PER_W
    pltpu.sync_copy(idx_hbm.at[wid], idx_v)

    def start(ci, b):
        pltpu.async_copy(h_hbm.at[idx_v.at[ci]], rows_v.at[b], gsems[b])

    def finish(ci, b):
        pltpu.make_async_copy(h_hbm.at[idx_v.at[ci]], rows_v.at[b], gsems[b]).wait()
        pltpu.sync_copy(rows_v.at[b], out_hbm.at[pl.ds(base + ci * C, C)])

    for b in range(NBUF - 1):
        start(b, b)

    def body(k, _):
        c0 = NBUF * k
        for b in range(NBUF):
            ci = c0 + b

            @pl.when(ci + NBUF - 1 < NCHUNK)
            def _():
                start(ci + NBUF - 1, (b + NBUF - 1) % NBUF)

            finish(ci, b)
        return 0

    lax.fori_loop(0, NCHUNK // NBUF, body, 0)


_sc_gather = functools.partial(
    pl.kernel,
    out_type=jax.ShapeDtypeStruct((E, HIDDEN), jnp.float32),
    mesh=plsc.VectorSubcoreMesh(core_axis_name="c", subcore_axis_name="s"),
    scratch_types=[
        pltpu.VMEM((NCHUNK, C), jnp.int32),
        pltpu.VMEM((NBUF, C, HIDDEN), jnp.float32),
    ] + [pltpu.SemaphoreType.DMA] * NBUF,
)(_sc_gather_body)


T = 640


def _tc_gru_body(x_ref, hnei_ref, wr_ref, ur_ref, urb_ref,
                 wzx_ref, wzh_ref, wzb_ref, whx_ref, whh_ref, whb_ref,
                 out_ref):
    xt = x_ref[...]
    r1 = jnp.dot(xt, wr_ref[...], preferred_element_type=jnp.float32)
    urb = urb_ref[...].reshape(1, HIDDEN)

    sum_h = jnp.zeros((T, HIDDEN), jnp.float32)
    sum_g = jnp.zeros((T, HIDDEN), jnp.float32)
    for j in range(MAX_NEI):
        hj = hnei_ref[j]
        r2 = jnp.dot(hj, ur_ref[...], preferred_element_type=jnp.float32)
        r = jax.nn.sigmoid(r1 + r2 + urb)
        sum_h = sum_h + hj
        sum_g = sum_g + r * hj

    z = jax.nn.sigmoid(
        jnp.dot(xt, wzx_ref[...], preferred_element_type=jnp.float32)
        + jnp.dot(sum_h, wzh_ref[...], preferred_element_type=jnp.float32)
        + wzb_ref[...].reshape(1, HIDDEN))
    pre_h = jnp.tanh(
        jnp.dot(xt, whx_ref[...], preferred_element_type=jnp.float32)
        + jnp.dot(sum_g, whh_ref[...], preferred_element_type=jnp.float32)
        + whb_ref[...].reshape(1, HIDDEN))
    h_new = (1.0 - z) * sum_h + z * pre_h

    row = lax.broadcasted_iota(jnp.int32, (T, HIDDEN), 0) + pl.program_id(0) * T
    out_ref[...] = jnp.where(row == 0, 0.0, h_new).astype(out_ref.dtype)


def _tc_gru(x, hnei, wr, ur, urb, wzx, wzh, wzb, whx, whh, whb, out_dtype):
    wspec = pl.BlockSpec((HIDDEN, HIDDEN), lambda i: (0, 0))
    bspec = pl.BlockSpec((HIDDEN,), lambda i: (0,))
    return pl.pallas_call(
        _tc_gru_body,
        grid=(N // T,),
        in_specs=[
            pl.BlockSpec((T, INPUT), lambda i: (i, 0)),
            pl.BlockSpec((MAX_NEI, T, HIDDEN), lambda i: (0, i, 0)),
            wspec, wspec, bspec, wspec, wspec, bspec, wspec, wspec, bspec,
        ],
        out_specs=pl.BlockSpec((T, HIDDEN), lambda i: (i, 0)),
        out_shape=jax.ShapeDtypeStruct((N, HIDDEN), out_dtype),
    )(x, hnei, wr, ur, urb, wzx, wzh, wzb, whx, whh, whb)


def kernel(h, x, mess_graph, W_z_w, W_z_b, W_r_w, U_r_w, U_r_b, W_h_w, W_h_b):
    wr = W_r_w.T
    ur = U_r_w.T
    wzx = W_z_w[:, :INPUT].T
    wzh = W_z_w[:, INPUT:].T
    whx = W_h_w[:, :INPUT].T
    whh = W_h_w[:, INPUT:].T
    idx3 = mess_graph.T.reshape(NW, NCHUNK, C)

    for _ in range(DEPTH):
        flat = _sc_gather(h, idx3)
        hnei = flat.reshape(MAX_NEI, N, HIDDEN)
        h = _tc_gru(x, hnei, wr, ur, U_r_b, wzx, wzh, W_z_b,
                    whx, whh, W_h_b, jnp.float32)
    return h


# speedup vs baseline: 4.7765x; 1.0974x over previous
import functools

import jax
import jax.numpy as jnp
from jax import lax
from jax.experimental import pallas as pl
from jax.experimental.pallas import tpu as pltpu
from jax.experimental.pallas import tpu_sc as plsc

N = 160000
MAX_NEI = 8
INPUT = 128
HIDDEN = 128
DEPTH = 3

SEGS = 5
SEG = N // SEGS

NC = 2
NS = 16
NW = NC * NS
ES = SEG * MAX_NEI
PER_W = ES // NW
C = 80
NCHUNK = PER_W // C
NBUF = 4


def _sc_gather_body(h_hbm, idx_hbm, out_hbm, idx_v, rows_v, *gsems):
    wid = lax.axis_index("s") * NC + lax.axis_index("c")
    base = wid * PER_W
    pltpu.sync_copy(idx_hbm.at[wid], idx_v)

    def start(ci, b):
        pltpu.async_copy(h_hbm.at[idx_v.at[ci]], rows_v.at[b], gsems[b])

    def finish(ci, b):
        pltpu.make_async_copy(h_hbm.at[idx_v.at[ci]], rows_v.at[b], gsems[b]).wait()
        pltpu.sync_copy(rows_v.at[b], out_hbm.at[pl.ds(base + ci * C, C)])

    for b in range(NBUF - 1):
        start(b, b)

    def body(k, _):
        c0 = NBUF * k
        for b in range(NBUF):
            ci = c0 + b

            @pl.when(ci + NBUF - 1 < NCHUNK)
            def _():
                start(ci + NBUF - 1, (b + NBUF - 1) % NBUF)

            finish(ci, b)
        return 0

    lax.fori_loop(0, NCHUNK // NBUF, body, 0)


_sc_gather = functools.partial(
    pl.kernel,
    out_type=jax.ShapeDtypeStruct((ES, HIDDEN), jnp.float32),
    mesh=plsc.VectorSubcoreMesh(core_axis_name="c", subcore_axis_name="s"),
    scratch_types=[
        pltpu.VMEM((NCHUNK, C), jnp.int32),
        pltpu.VMEM((NBUF, C, HIDDEN), jnp.float32),
    ] + [pltpu.SemaphoreType.DMA] * NBUF,
)(_sc_gather_body)


T = 640


def _tc_gru_body(hacc_ref, x_ref, hnei_ref, wr_ref, ur_ref, urb_ref,
                 wzx_ref, wzh_ref, wzb_ref, whx_ref, whh_ref, whb_ref,
                 out_ref, seg):
    del hacc_ref
    xt = x_ref[...]
    r1 = jnp.dot(xt, wr_ref[...], preferred_element_type=jnp.float32)
    urb = urb_ref[...].reshape(1, HIDDEN)

    sum_h = jnp.zeros((T, HIDDEN), jnp.float32)
    sum_g = jnp.zeros((T, HIDDEN), jnp.float32)
    for j in range(MAX_NEI):
        hj = hnei_ref[j]
        r2 = jnp.dot(hj, ur_ref[...], preferred_element_type=jnp.float32)
        r = jax.nn.sigmoid(r1 + r2 + urb)
        sum_h = sum_h + hj
        sum_g = sum_g + r * hj

    z = jax.nn.sigmoid(
        jnp.dot(xt, wzx_ref[...], preferred_element_type=jnp.float32)
        + jnp.dot(sum_h, wzh_ref[...], preferred_element_type=jnp.float32)
        + wzb_ref[...].reshape(1, HIDDEN))
    pre_h = jnp.tanh(
        jnp.dot(xt, whx_ref[...], preferred_element_type=jnp.float32)
        + jnp.dot(sum_g, whh_ref[...], preferred_element_type=jnp.float32)
        + whb_ref[...].reshape(1, HIDDEN))
    h_new = (1.0 - z) * sum_h + z * pre_h

    row = (lax.broadcasted_iota(jnp.int32, (T, HIDDEN), 0)
           + (seg * SEG + pl.program_id(0) * T))
    out_ref[...] = jnp.where(row == 0, 0.0, h_new)


def _tc_gru_seg(seg, h_acc, x, hnei, weights):
    t0 = seg * (SEG // T)
    wspec = pl.BlockSpec((HIDDEN, HIDDEN), lambda i: (0, 0))
    bspec = pl.BlockSpec((HIDDEN,), lambda i: (0,))
    return pl.pallas_call(
        functools.partial(_tc_gru_body, seg=seg),
        grid=(SEG // T,),
        in_specs=[
            pl.BlockSpec(memory_space=pltpu.HBM),
            pl.BlockSpec((T, INPUT), lambda i: (t0 + i, 0)),
            pl.BlockSpec((MAX_NEI, T, HIDDEN), lambda i: (0, i, 0)),
            wspec, wspec, bspec, wspec, wspec, bspec, wspec, wspec, bspec,
        ],
        out_specs=pl.BlockSpec((T, HIDDEN), lambda i: (t0 + i, 0)),
        out_shape=jax.ShapeDtypeStruct((N, HIDDEN), jnp.float32),
        input_output_aliases={0: 0},
    )(h_acc, x, hnei, *weights)


def kernel(h, x, mess_graph, W_z_w, W_z_b, W_r_w, U_r_w, U_r_b, W_h_w, W_h_b):
    wr = W_r_w.T
    ur = U_r_w.T
    wzx = W_z_w[:, :INPUT].T
    wzh = W_z_w[:, INPUT:].T
    whx = W_h_w[:, :INPUT].T
    whh = W_h_w[:, INPUT:].T
    weights = (wr, ur, U_r_b, wzx, wzh, W_z_b, whx, whh, W_h_b)
    idx = [mess_graph[s * SEG:(s + 1) * SEG].T.reshape(NW, NCHUNK, C)
           for s in range(SEGS)]

    bufs = [jnp.zeros((N, HIDDEN), jnp.float32) for _ in range(2)]
    hist = []
    for d in range(DEPTH):
        acc = bufs[d] if d < 2 else hist[d - 2]
        for s in range(SEGS):
            flat = _sc_gather(h, idx[s])
            hnei = flat.reshape(MAX_NEI, SEG, HIDDEN)
            acc = _tc_gru_seg(s, acc, x, hnei, weights)
        hist.append(acc)
        h = acc
    return h
